# baseline (speedup 1.0000x reference)
import functools

import numpy as np

import jax
import jax.numpy as jnp
from jax import lax
from jax.experimental import pallas as pl
from jax.experimental.pallas import tpu as pltpu
from jax.experimental.pallas import tpu_sc as plsc

N_USERS = 4000
N_ITEMS = 6000
N = N_USERS + N_ITEMS
D = 128
E = 320000
N_LAYERS = 2

NC = 2
NS = 16
NW = NC * NS
K = 32
NBUF = 2
CH = 316
E_PAD = NW * CH * K
WR = CH * K // 128
PR = WR + 1
ROWS_PER_TILE = 632
N_PAD = NS * ROWS_PER_TILE
_OUT_CHUNKS = (128, 128, 128, 128, 120)


_sc_mesh = plsc.VectorSubcoreMesh(core_axis_name="c", subcore_axis_name="s")


@functools.partial(
    pl.kernel,
    out_type=jax.ShapeDtypeStruct((NC * N_PAD, D), jnp.float32),
    mesh=_sc_mesh,
    scratch_types=(
        [pltpu.VMEM((PR, 128), jnp.int32),
         pltpu.VMEM((WR, 128), jnp.float32)]
        + [pltpu.VMEM((K, D), jnp.float32)] * (2 * NBUF)
        + [pltpu.VMEM((K,), jnp.int32)] * (2 * NBUF)
        + [pltpu.VMEM_SHARED((N_PAD, D), jnp.float32)]
        + [pltpu.SemaphoreType.DMA] * (2 * NBUF)
    ),
)
def _propagate(table, packed3, w3, out, packed_v, w_v, *rest):
    gs = rest[0:NBUF]
    ss = rest[NBUF:2 * NBUF]
    sis = rest[2 * NBUF:3 * NBUF]
    dis = rest[3 * NBUF:4 * NBUF]
    acc = rest[4 * NBUF]
    gsems = rest[4 * NBUF + 1:5 * NBUF + 1]
    ssems = rest[5 * NBUF + 1:6 * NBUF + 1]
    c = lax.axis_index("c")
    s = lax.axis_index("s")
    wid = s * NC + c
    bufs = tuple((gs[b], ss[b], sis[b], dis[b], gsems[b], ssems[b])
                 for b in range(NBUF))
    s_a = ss[0]
    gsem_a, gsem_b, ssem_a = gsems[0], gsems[1], ssems[0]

    pltpu.async_copy(packed3.at[wid], packed_v, gsem_a)
    pltpu.async_copy(w3.at[wid], w_v, gsem_b)

    zero16 = jnp.zeros((16,), jnp.float32)

    def zrow(i, carry):
        for dd in range(D // 16):
            s_a[i, pl.ds(dd * 16, 16)] = zero16
        return carry

    lax.fori_loop(0, K, zrow, 0)
    row0 = s * ROWS_PER_TILE
    for zi in range(ROWS_PER_TILE // K):
        pltpu.async_copy(s_a, acc.at[pl.ds(row0 + zi * K, K)], ssem_a)
    rem = ROWS_PER_TILE % K
    if rem:
        pltpu.async_copy(s_a.at[pl.ds(0, rem)],
                         acc.at[pl.ds(row0 + ROWS_PER_TILE - rem, rem)],
                         ssem_a)
    for zi in range(ROWS_PER_TILE // K):
        pltpu.make_async_copy(s_a, acc.at[pl.ds(row0, K)], ssem_a).wait()
    if rem:
        pltpu.make_async_copy(s_a.at[pl.ds(0, rem)],
                              acc.at[pl.ds(row0, rem)], ssem_a).wait()
    pltpu.make_async_copy(packed3.at[wid], packed_v, gsem_a).wait()
    pltpu.make_async_copy(w3.at[wid], w_v, gsem_b).wait()
    plsc.subcore_barrier()

    def scale(gb, sb, j):
        r = j // 4
        c0 = (j % 4) * K

        def group(g, carry):
            w16 = w_v[r, pl.ds(c0 + g * 16, 16)]
            e0 = g * 16
            for i in range(16):
                splat = jnp.full((16,), w16[i], jnp.float32)
                for dd in range(D // 16):
                    sb[e0 + i, pl.ds(dd * 16, 16)] = (
                        gb[e0 + i, pl.ds(dd * 16, 16)] * splat)
            return carry

        lax.fori_loop(0, K // 16, group, 0)

    def extract_src(b, j):
        gb, sb, si, di, gsem, ssem = bufs[b]
        r = j // 4
        c0 = (j % 4) * K
        for kk in range(K // 16):
            p16 = packed_v[r, pl.ds(c0 + kk * 16, 16)]
            si[pl.ds(kk * 16, 16)] = lax.shift_right_logical(p16, 14)

    def extract_dst(b, j):
        gb, sb, si, di, gsem, ssem = bufs[b]
        r = j // 4
        c0 = (j % 4) * K
        for kk in range(K // 16):
            p16 = packed_v[r, pl.ds(c0 + kk * 16, 16)]
            di[pl.ds(kk * 16, 16)] = lax.bitwise_and(p16, 16383)

    def fire_gather(b, j):
        gb, sb, si, di, gsem, ssem = bufs[b]
        pltpu.async_copy(table.at[si], gb, gsem)

    def fire_scatter(b, j):
        gb, sb, si, di, gsem, ssem = bufs[b]
        pltpu.async_copy(sb, acc.at[di], ssem, add=True)

    for b in range(NBUF):
        extract_src(b, b)
    for b in range(NBUF):
        fire_gather(b, b)
    for b in range(NBUF):
        gb, sb, si, di, gsem, ssem = bufs[b]
        extract_dst(b, b)
        pltpu.make_async_copy(table.at[si], gb, gsem).wait()
        extract_src(b, b + NBUF)
        scale(gb, sb, b)
        fire_gather(b, b + NBUF)
        fire_scatter(b, b)

    def wave(jj, carry):
        for b in range(NBUF):
            j = jj * NBUF + b
            gb, sb, si, di, gsem, ssem = bufs[b]
            pltpu.make_async_copy(sb, acc.at[di], ssem).wait()
            extract_dst(b, j)
            pltpu.make_async_copy(table.at[si], gb, gsem).wait()
            extract_src(b, j + NBUF)
            scale(gb, sb, j)
            fire_gather(b, j + NBUF)
            fire_scatter(b, j)
        return carry

    lax.fori_loop(1, CH // NBUF, wave, 0)
    for b in range(NBUF):
        gb, sb, si, di, gsem, ssem = bufs[b]
        pltpu.make_async_copy(sb, acc.at[di], ssem).wait()
        pltpu.make_async_copy(table.at[si], gb, gsem).wait()
    plsc.subcore_barrier()

    off = 0
    for sz in _OUT_CHUNKS:
        pltpu.async_copy(acc.at[pl.ds(row0 + off, sz)],
                         out.at[pl.ds(c * N_PAD + row0 + off, sz)], gsem_a)
        off += sz
    for sz in _OUT_CHUNKS:
        pltpu.make_async_copy(acc.at[pl.ds(row0, sz)],
                              out.at[pl.ds(c * N_PAD + row0, sz)],
                              gsem_a).wait()


def _add_body(a_ref, b_ref, o_ref):
    o_ref[...] = a_ref[...] + b_ref[...]


_combine = pl.pallas_call(
    _add_body,
    out_shape=jax.ShapeDtypeStruct((N_PAD, D), jnp.float32),
    grid=(NS,),
    in_specs=[
        pl.BlockSpec((None, ROWS_PER_TILE, D), lambda i: (0, i, 0)),
        pl.BlockSpec((None, ROWS_PER_TILE, D), lambda i: (1, i, 0)),
    ],
    out_specs=pl.BlockSpec((ROWS_PER_TILE, D), lambda i: (i, 0)),
)


def _pack_body(c0_ref, c1_ref, p2a_ref, p2b_ref, con_ref, bin_ref):
    xs = (c0_ref[...], c1_ref[...], p2a_ref[...] + p2b_ref[...])
    for k3, x in enumerate(xs):
        lam = (k3 + 1) / (N_LAYERS + 1)
        con_ref[:, k3 * D:(k3 + 1) * D] = x
        m = jnp.sum(jnp.abs(x), axis=1, keepdims=True) * (lam / D)
        bin_ref[:, k3 * D:(k3 + 1) * D] = jnp.sign(x) * m


def _make_pack(base_blk, rows):
    nb = rows // 1000
    return pl.pallas_call(
        _pack_body,
        out_shape=(
            jax.ShapeDtypeStruct((rows, 3 * D), jnp.float32),
            jax.ShapeDtypeStruct((rows, 3 * D), jnp.float32),
        ),
        grid=(nb,),
        in_specs=[
            pl.BlockSpec((1000, D), lambda i: (base_blk + i, 0)),
            pl.BlockSpec((1000, D), lambda i: (base_blk + i, 0)),
            pl.BlockSpec((None, 1000, D), lambda i: (0, base_blk + i, 0)),
            pl.BlockSpec((None, 1000, D), lambda i: (1, base_blk + i, 0)),
        ],
        out_specs=(
            pl.BlockSpec((1000, 3 * D), lambda i: (i, 0)),
            pl.BlockSpec((1000, 3 * D), lambda i: (i, 0)),
        ),
    )


_pack_users = _make_pack(0, N_USERS)
_pack_items = _make_pack(N_USERS // 1000, N_ITEMS)


def kernel(user_embed, item_embed, edge_weight, edge_index):
    con0 = jnp.concatenate([user_embed, item_embed], axis=0)
    pad = E_PAD - E
    packed = (edge_index[0] << 14) | edge_index[1]
    packed3 = jnp.pad(jnp.pad(packed, (0, pad)).reshape(NW, WR, 128),
                      ((0, 0), (0, 1), (0, 0)))
    w3 = jnp.pad(edge_weight, (0, pad)).reshape(NW, WR, 128)

    p1 = _propagate(con0, packed3, w3).reshape(NC, N_PAD, D)
    con1 = _combine(p1, p1)
    p2 = _propagate(con1, packed3, w3).reshape(NC, N_PAD, D)

    con_u, bin_u = _pack_users(con0, con1, p2, p2)
    con_i, bin_i = _pack_items(con0, con1, p2, p2)
    return (con_u, con_i, bin_u, bin_i)


# speedup vs baseline: 1.0004x; 1.0004x over previous
import functools

import jax
import jax.numpy as jnp
from jax import lax
from jax.experimental import pallas as pl
from jax.experimental.pallas import tpu as pltpu
from jax.experimental.pallas import tpu_sc as plsc

N_USERS = 4000
N_ITEMS = 6000
N = N_USERS + N_ITEMS
D = 128
E = 320000
N_LAYERS = 2

NC = 2
NS = 16
NW = NC * NS
K = 32
NBUF = 2
CH = 316
E_PAD = NW * CH * K
WR = CH * K // 128
PR = WR + 1
ROWS_PER_TILE = 632
N_PAD = NS * ROWS_PER_TILE
_OUT_CHUNKS = (128, 128, 128, 128, 120)


_sc_mesh = plsc.VectorSubcoreMesh(core_axis_name="c", subcore_axis_name="s")


@functools.partial(
    pl.kernel,
    out_type=jax.ShapeDtypeStruct((NC * N_PAD, D), jnp.float32),
    mesh=_sc_mesh,
    scratch_types=(
        [pltpu.VMEM((PR, 128), jnp.int32),
         pltpu.VMEM((WR, 128), jnp.float32)]
        + [pltpu.VMEM((K, D), jnp.float32)] * (2 * NBUF)
        + [pltpu.VMEM((K,), jnp.int32)] * (2 * NBUF)
        + [pltpu.VMEM_SHARED((N_PAD, D), jnp.float32)]
        + [pltpu.SemaphoreType.DMA] * (2 * NBUF)
    ),
)
def _propagate(table, packed3, w3, out, packed_v, w_v, *rest):
    gs = rest[0:NBUF]
    ss = rest[NBUF:2 * NBUF]
    sis = rest[2 * NBUF:3 * NBUF]
    dis = rest[3 * NBUF:4 * NBUF]
    acc = rest[4 * NBUF]
    gsems = rest[4 * NBUF + 1:5 * NBUF + 1]
    ssems = rest[5 * NBUF + 1:6 * NBUF + 1]
    c = lax.axis_index("c")
    s = lax.axis_index("s")
    wid = s * NC + c
    bufs = tuple((gs[b], ss[b], sis[b], dis[b], gsems[b], ssems[b])
                 for b in range(NBUF))
    s_a = ss[0]
    gsem_a, gsem_b, ssem_a = gsems[0], gsems[1], ssems[0]

    pltpu.async_copy(packed3.at[wid], packed_v, gsem_a)
    pltpu.async_copy(w3.at[wid], w_v, gsem_b)

    zero16 = jnp.zeros((16,), jnp.float32)

    def zrow(i, carry):
        for dd in range(D // 16):
            s_a[i, pl.ds(dd * 16, 16)] = zero16
        return carry

    lax.fori_loop(0, K, zrow, 0)
    row0 = s * ROWS_PER_TILE
    for zi in range(ROWS_PER_TILE // K):
        pltpu.async_copy(s_a, acc.at[pl.ds(row0 + zi * K, K)], ssem_a)
    rem = ROWS_PER_TILE % K
    if rem:
        pltpu.async_copy(s_a.at[pl.ds(0, rem)],
                         acc.at[pl.ds(row0 + ROWS_PER_TILE - rem, rem)],
                         ssem_a)
    for zi in range(ROWS_PER_TILE // K):
        pltpu.make_async_copy(s_a, acc.at[pl.ds(row0, K)], ssem_a).wait()
    if rem:
        pltpu.make_async_copy(s_a.at[pl.ds(0, rem)],
                              acc.at[pl.ds(row0, rem)], ssem_a).wait()
    pltpu.make_async_copy(packed3.at[wid], packed_v, gsem_a).wait()
    pltpu.make_async_copy(w3.at[wid], w_v, gsem_b).wait()
    plsc.subcore_barrier()

    def scale(gb, sb, j):
        r = j // 4
        c0 = (j % 4) * K

        def group(g, carry):
            w16 = w_v[r, pl.ds(c0 + g * 16, 16)]
            e0 = g * 16
            for i in range(16):
                splat = jnp.full((16,), w16[i], jnp.float32)
                for dd in range(D // 16):
                    sb[e0 + i, pl.ds(dd * 16, 16)] = (
                        gb[e0 + i, pl.ds(dd * 16, 16)] * splat)
            return carry

        lax.fori_loop(0, K // 16, group, 0)

    def extract_src(b, j):
        gb, sb, si, di, gsem, ssem = bufs[b]
        r = j // 4
        c0 = (j % 4) * K
        for kk in range(K // 16):
            p16 = packed_v[r, pl.ds(c0 + kk * 16, 16)]
            si[pl.ds(kk * 16, 16)] = lax.shift_right_logical(p16, 14)

    def extract_dst(b, j):
        gb, sb, si, di, gsem, ssem = bufs[b]
        r = j // 4
        c0 = (j % 4) * K
        for kk in range(K // 16):
            p16 = packed_v[r, pl.ds(c0 + kk * 16, 16)]
            di[pl.ds(kk * 16, 16)] = lax.bitwise_and(p16, 16383)

    def fire_gather(b, j):
        gb, sb, si, di, gsem, ssem = bufs[b]
        pltpu.async_copy(table.at[si], gb, gsem)

    def fire_scatter(b, j):
        gb, sb, si, di, gsem, ssem = bufs[b]
        pltpu.async_copy(sb, acc.at[di], ssem, add=True)

    for b in range(NBUF):
        extract_src(b, b)
    for b in range(NBUF):
        fire_gather(b, b)
    for b in range(NBUF):
        gb, sb, si, di, gsem, ssem = bufs[b]
        extract_dst(b, b)
        pltpu.make_async_copy(table.at[si], gb, gsem).wait()
        extract_src(b, b + NBUF)
        scale(gb, sb, b)
        fire_gather(b, b + NBUF)
        fire_scatter(b, b)

    def wave(jj, carry):
        for b in range(NBUF):
            j = jj * NBUF + b
            gb, sb, si, di, gsem, ssem = bufs[b]
            pltpu.make_async_copy(sb, acc.at[di], ssem).wait()
            extract_dst(b, j)
            pltpu.make_async_copy(table.at[si], gb, gsem).wait()
            extract_src(b, j + NBUF)
            scale(gb, sb, j)
            fire_gather(b, j + NBUF)
            fire_scatter(b, j)
        return carry

    lax.fori_loop(1, CH // NBUF, wave, 0)
    for b in range(NBUF):
        gb, sb, si, di, gsem, ssem = bufs[b]
        pltpu.make_async_copy(sb, acc.at[di], ssem).wait()
        pltpu.make_async_copy(table.at[si], gb, gsem).wait()
    plsc.subcore_barrier()

    off = 0
    for sz in _OUT_CHUNKS:
        pltpu.async_copy(acc.at[pl.ds(row0 + off, sz)],
                         out.at[pl.ds(c * N_PAD + row0 + off, sz)], gsem_a)
        off += sz
    for sz in _OUT_CHUNKS:
        pltpu.make_async_copy(acc.at[pl.ds(row0, sz)],
                              out.at[pl.ds(c * N_PAD + row0, sz)],
                              gsem_a).wait()


def _add_body(a_ref, b_ref, o_ref):
    o_ref[...] = a_ref[...] + b_ref[...]


_combine = pl.pallas_call(
    _add_body,
    out_shape=jax.ShapeDtypeStruct((N_PAD, D), jnp.float32),
    grid=(NS,),
    in_specs=[
        pl.BlockSpec((None, ROWS_PER_TILE, D), lambda i: (0, i, 0)),
        pl.BlockSpec((None, ROWS_PER_TILE, D), lambda i: (1, i, 0)),
    ],
    out_specs=pl.BlockSpec((ROWS_PER_TILE, D), lambda i: (i, 0)),
)


def _pack_body(c0_ref, c1_ref, p2a_ref, p2b_ref, con_ref, bin_ref):
    xs = (c0_ref[...], c1_ref[...], p2a_ref[...] + p2b_ref[...])
    for k3, x in enumerate(xs):
        lam = (k3 + 1) / (N_LAYERS + 1)
        con_ref[:, k3 * D:(k3 + 1) * D] = x
        m = jnp.sum(jnp.abs(x), axis=1, keepdims=True) * (lam / D)
        bin_ref[:, k3 * D:(k3 + 1) * D] = jnp.sign(x) * m


def _make_pack(base_blk, rows):
    nb = rows // 1000
    return pl.pallas_call(
        _pack_body,
        out_shape=(
            jax.ShapeDtypeStruct((rows, 3 * D), jnp.float32),
            jax.ShapeDtypeStruct((rows, 3 * D), jnp.float32),
        ),
        grid=(nb,),
        in_specs=[
            pl.BlockSpec((1000, D), lambda i: (base_blk + i, 0)),
            pl.BlockSpec((1000, D), lambda i: (base_blk + i, 0)),
            pl.BlockSpec((None, 1000, D), lambda i: (0, base_blk + i, 0)),
            pl.BlockSpec((None, 1000, D), lambda i: (1, base_blk + i, 0)),
        ],
        out_specs=(
            pl.BlockSpec((1000, 3 * D), lambda i: (i, 0)),
            pl.BlockSpec((1000, 3 * D), lambda i: (i, 0)),
        ),
    )


_pack_users = _make_pack(0, N_USERS)
_pack_items = _make_pack(N_USERS // 1000, N_ITEMS)


def kernel(user_embed, item_embed, edge_weight, edge_index):
    con0 = jnp.concatenate([user_embed, item_embed], axis=0)
    pad = E_PAD - E
    packed = (edge_index[0] << 14) | edge_index[1]
    packed3 = jnp.pad(jnp.pad(packed, (0, pad)).reshape(NW, WR, 128),
                      ((0, 0), (0, 1), (0, 0)))
    w3 = jnp.pad(edge_weight, (0, pad)).reshape(NW, WR, 128)

    p1 = _propagate(con0, packed3, w3).reshape(NC, N_PAD, D)
    con1 = _combine(p1, p1)
    p2 = _propagate(con1, packed3, w3).reshape(NC, N_PAD, D)

    con_u, bin_u = _pack_users(con0, con1, p2, p2)
    con_i, bin_i = _pack_items(con0, con1, p2, p2)
    return (con_u, con_i, bin_u, bin_i)
